# Initial kernel scaffold; baseline (speedup 1.0000x reference)
import jax
import jax.numpy as jnp
from jax.experimental import pallas as pl


def kernel(x, gamma):
    raise NotImplementedError("write your pallas kernel here")


# speedup vs baseline: 4.2831x; 4.2831x over previous
import jax
import jax.numpy as jnp
from jax.experimental import pallas as pl
from jax.experimental.pallas import tpu as pltpu

_BS = 5
_CT = 8


def _dilated_mask(gamma, step, ct, h, w):
    pltpu.prng_seed(step)
    bits = pltpu.bitcast(pltpu.prng_random_bits((ct, h, w)), jnp.uint32)
    u = (bits >> 8).astype(jnp.float32) * (1.0 / (1 << 24))
    vh = jax.lax.broadcasted_iota(jnp.int32, (ct, h, w), 1) < (h - (_BS - 1))
    vw = jax.lax.broadcasted_iota(jnp.int32, (ct, h, w), 2) < (w - (_BS - 1))
    m = jnp.where((u < gamma) & vh & vw, 1.0, 0.0)
    zh = jnp.zeros((ct, _BS - 1, w), jnp.float32)
    mp = jnp.concatenate([zh, m], axis=1)
    r = m
    for i in range(_BS - 1):
        r = jnp.maximum(r, mp[:, i:i + h, :])
    zw = jnp.zeros((ct, h, _BS - 1), jnp.float32)
    rp = jnp.concatenate([zw, r], axis=2)
    d = r
    for i in range(_BS - 1):
        d = jnp.maximum(d, rp[:, :, i:i + w])
    return d


def _count_body(h, w):
    def body(gamma_ref, out_ref, acc_ref):
        step = pl.program_id(0)
        d = _dilated_mask(gamma_ref[0, 0], step, _CT, h, w)
        part = jnp.sum(d)

        @pl.when(step == 0)
        def _():
            acc_ref[0, 0] = 0.0

        acc_ref[0, 0] += part

        @pl.when(step == pl.num_programs(0) - 1)
        def _():
            out_ref[0, 0] = acc_ref[0, 0]

    return body


def _apply_body(h, w, count_m):
    def body(gamma_ref, cnt_ref, x_ref, out_ref):
        step = pl.program_id(0)
        d = _dilated_mask(gamma_ref[0, 0], step, _CT, h, w)
        scale = count_m / (count_m - cnt_ref[0, 0])
        out_ref[...] = x_ref[...] * ((1.0 - d) * scale)

    return body


def kernel(x, gamma):
    b, c, h, w = x.shape
    n = b * c
    grid = (n // _CT,)
    xf = x.reshape(n, h, w)
    g = gamma.reshape(1, 1).astype(jnp.float32)
    count_m = float(b * c * h * w)

    sum_dilated = pl.pallas_call(
        _count_body(h, w),
        grid=grid,
        in_specs=[pl.BlockSpec(memory_space=pltpu.SMEM)],
        out_specs=pl.BlockSpec(memory_space=pltpu.SMEM),
        out_shape=jax.ShapeDtypeStruct((1, 1), jnp.float32),
        scratch_shapes=[pltpu.SMEM((1, 1), jnp.float32)],
    )(g)

    out = pl.pallas_call(
        _apply_body(h, w, count_m),
        grid=grid,
        in_specs=[
            pl.BlockSpec(memory_space=pltpu.SMEM),
            pl.BlockSpec(memory_space=pltpu.SMEM),
            pl.BlockSpec((_CT, h, w), lambda i: (i, 0, 0)),
        ],
        out_specs=pl.BlockSpec((_CT, h, w), lambda i: (i, 0, 0)),
        out_shape=jax.ShapeDtypeStruct((n, h, w), jnp.float32),
    )(g, sum_dilated, xf)

    return out.reshape(b, c, h, w)


# speedup vs baseline: 6.5298x; 1.5246x over previous
import jax
import jax.numpy as jnp
from jax.experimental import pallas as pl
from jax.experimental.pallas import tpu as pltpu

_BS = 5
_CT = 16
_PCORES = 2


def _shift_h(a, k, ct, h, w):
    z = jnp.zeros((ct, k, w), jnp.float32)
    return jnp.concatenate([z, a[:, :h - k, :]], axis=1)


def _shift_w(a, k, ct, h, w):
    z = jnp.zeros((ct, h, k), jnp.float32)
    return jnp.concatenate([z, a[:, :, :w - k]], axis=2)


def _dilated_mask(gamma, group, ct, h, w):
    pltpu.prng_seed(group)
    bits = pltpu.bitcast(pltpu.prng_random_bits((ct, h, w)), jnp.uint32)
    b31 = (bits >> jnp.uint32(1)).astype(jnp.int32)
    thresh = (jnp.clip(gamma, 0.0, 1.0) * 2147483647.0).astype(jnp.int32)
    vh = jax.lax.broadcasted_iota(jnp.int32, (ct, h, w), 1) < (h - (_BS - 1))
    vw = jax.lax.broadcasted_iota(jnp.int32, (ct, h, w), 2) < (w - (_BS - 1))
    m = jnp.where((b31 < thresh) & vh & vw, 1.0, 0.0)
    t = jnp.maximum(m, _shift_h(m, 1, ct, h, w))
    t = jnp.maximum(t, _shift_h(t, 2, ct, h, w))
    r = jnp.maximum(t, _shift_h(m, 4, ct, h, w))
    t = jnp.maximum(r, _shift_w(r, 1, ct, h, w))
    t = jnp.maximum(t, _shift_w(t, 2, ct, h, w))
    d = jnp.maximum(t, _shift_w(r, 4, ct, h, w))
    return d


def _count_body(h, w, inner):
    def body(gamma_ref, out_ref, acc_ref):
        p = pl.program_id(0)
        s = pl.program_id(1)
        d = _dilated_mask(gamma_ref[0, 0], p * inner + s, _CT, h, w)
        part = jnp.sum(d)

        @pl.when(s == 0)
        def _():
            acc_ref[0, 0] = 0.0

        acc_ref[0, 0] += part

        @pl.when(s == inner - 1)
        def _():
            out_ref[p, 0] = acc_ref[0, 0]

    return body


def _apply_body(h, w, count_m):
    def body(gamma_ref, cnt_ref, x_ref, out_ref):
        group = pl.program_id(0)
        d = _dilated_mask(gamma_ref[0, 0], group, _CT, h, w)
        sum_dilated = cnt_ref[0, 0] + cnt_ref[1, 0]
        scale = count_m / (count_m - sum_dilated)
        out_ref[...] = jnp.where(d > 0.5, 0.0, x_ref[...] * scale)

    return body


def kernel(x, gamma):
    b, c, h, w = x.shape
    n = b * c
    groups = n // _CT
    inner = groups // _PCORES
    xf = x.reshape(n, h, w)
    g = gamma.reshape(1, 1).astype(jnp.float32)
    count_m = float(b * c * h * w)

    partials = pl.pallas_call(
        _count_body(h, w, inner),
        grid=(_PCORES, inner),
        in_specs=[pl.BlockSpec(memory_space=pltpu.SMEM)],
        out_specs=pl.BlockSpec(memory_space=pltpu.SMEM),
        out_shape=jax.ShapeDtypeStruct((_PCORES, 1), jnp.float32),
        scratch_shapes=[pltpu.SMEM((1, 1), jnp.float32)],
        compiler_params=pltpu.CompilerParams(
            dimension_semantics=("parallel", "arbitrary"),
        ),
    )(g)

    out = pl.pallas_call(
        _apply_body(h, w, count_m),
        grid=(groups,),
        in_specs=[
            pl.BlockSpec(memory_space=pltpu.SMEM),
            pl.BlockSpec(memory_space=pltpu.SMEM),
            pl.BlockSpec((_CT, h, w), lambda i: (i, 0, 0)),
        ],
        out_specs=pl.BlockSpec((_CT, h, w), lambda i: (i, 0, 0)),
        out_shape=jax.ShapeDtypeStruct((n, h, w), jnp.float32),
        compiler_params=pltpu.CompilerParams(
            dimension_semantics=("parallel",),
        ),
    )(g, partials, xf)

    return out.reshape(b, c, h, w)


# speedup vs baseline: 6.5470x; 1.0026x over previous
import jax
import jax.numpy as jnp
from jax.experimental import pallas as pl
from jax.experimental.pallas import tpu as pltpu

_BS = 5
_CT = 16
_PCORES = 2


def _shift_h(a, k, ct, h, w):
    z = jnp.zeros((ct, k, w), jnp.float32)
    return jnp.concatenate([z, a[:, :h - k, :]], axis=1)


def _shift_w(a, k, ct, h, w):
    z = jnp.zeros((ct, h, k), jnp.float32)
    return jnp.concatenate([z, a[:, :, :w - k]], axis=2)


def _dilated_mask(gamma, group, ct, h, w):
    hv = h - (_BS - 1)
    pltpu.prng_seed(group)
    bits = pltpu.bitcast(pltpu.prng_random_bits((ct, hv, w)), jnp.uint32)
    b31 = (bits >> jnp.uint32(1)).astype(jnp.int32)
    thresh = (jnp.clip(gamma, 0.0, 1.0) * 2147483647.0).astype(jnp.int32)
    lane = jax.lax.broadcasted_iota(jnp.int32, (1, 1, w), 2)
    tvec = jnp.where(lane < (w - (_BS - 1)), thresh, 0)
    m = jnp.where(b31 < tvec, 1.0, 0.0)
    mu = jnp.concatenate([m, jnp.zeros((ct, h - hv, w), jnp.float32)], axis=1)
    t = jnp.maximum(mu, _shift_h(mu, 1, ct, h, w))
    t = jnp.maximum(t, _shift_h(t, 2, ct, h, w))
    r = jnp.maximum(t, _shift_h(mu, 4, ct, h, w))
    t = jnp.maximum(r, _shift_w(r, 1, ct, h, w))
    t = jnp.maximum(t, _shift_w(t, 2, ct, h, w))
    d = jnp.maximum(t, _shift_w(r, 4, ct, h, w))
    return d


def _count_body(h, w, inner):
    def body(gamma_ref, out_ref, acc_ref):
        p = pl.program_id(0)
        s = pl.program_id(1)
        d = _dilated_mask(gamma_ref[0, 0], p * inner + s, _CT, h, w)
        part = jnp.sum(d)

        @pl.when(s == 0)
        def _():
            acc_ref[0, 0] = 0.0

        acc_ref[0, 0] += part

        @pl.when(s == inner - 1)
        def _():
            out_ref[p, 0] = acc_ref[0, 0]

    return body


def _apply_body(h, w, count_m):
    def body(gamma_ref, cnt_ref, x_ref, out_ref):
        group = pl.program_id(0)
        d = _dilated_mask(gamma_ref[0, 0], group, _CT, h, w)
        sum_dilated = cnt_ref[0, 0] + cnt_ref[1, 0]
        scale = count_m / (count_m - sum_dilated)
        out_ref[...] = jnp.where(d > 0.5, 0.0, x_ref[...] * scale)

    return body


def kernel(x, gamma):
    b, c, h, w = x.shape
    n = b * c
    groups = n // _CT
    inner = groups // _PCORES
    xf = x.reshape(n, h, w)
    g = gamma.reshape(1, 1).astype(jnp.float32)
    count_m = float(b * c * h * w)

    partials = pl.pallas_call(
        _count_body(h, w, inner),
        grid=(_PCORES, inner),
        in_specs=[pl.BlockSpec(memory_space=pltpu.SMEM)],
        out_specs=pl.BlockSpec(memory_space=pltpu.SMEM),
        out_shape=jax.ShapeDtypeStruct((_PCORES, 1), jnp.float32),
        scratch_shapes=[pltpu.SMEM((1, 1), jnp.float32)],
        compiler_params=pltpu.CompilerParams(
            dimension_semantics=("parallel", "arbitrary"),
        ),
    )(g)

    out = pl.pallas_call(
        _apply_body(h, w, count_m),
        grid=(groups,),
        in_specs=[
            pl.BlockSpec(memory_space=pltpu.SMEM),
            pl.BlockSpec(memory_space=pltpu.SMEM),
            pl.BlockSpec((_CT, h, w), lambda i: (i, 0, 0)),
        ],
        out_specs=pl.BlockSpec((_CT, h, w), lambda i: (i, 0, 0)),
        out_shape=jax.ShapeDtypeStruct((n, h, w), jnp.float32),
        compiler_params=pltpu.CompilerParams(
            dimension_semantics=("parallel",),
        ),
    )(g, partials, xf)

    return out.reshape(b, c, h, w)


# speedup vs baseline: 8.3626x; 1.2773x over previous
import jax
import jax.numpy as jnp
from jax.experimental import pallas as pl
from jax.experimental.pallas import tpu as pltpu

_BS = 5
_ST = 16
_CTA = 64
_CTC = 64
_PCORES = 2


def _shift_h(a, k, ct, h, w):
    z = jnp.zeros((ct, k, w), jnp.float32)
    return jnp.concatenate([z, a[:, :h - k, :]], axis=1)


def _shift_w(a, k, ct, h, w):
    z = jnp.zeros((ct, h, k), jnp.float32)
    return jnp.concatenate([z, a[:, :, :w - k]], axis=2)


def _dilated_mask(gamma, seed_idx, h, w):
    ct = _ST
    hv = h - (_BS - 1)
    pltpu.prng_seed(seed_idx)
    bits = pltpu.bitcast(pltpu.prng_random_bits((ct, hv, w)), jnp.uint32)
    b31 = (bits >> jnp.uint32(1)).astype(jnp.int32)
    thresh = (jnp.clip(gamma, 0.0, 1.0) * 2147483647.0).astype(jnp.int32)
    lane = jax.lax.broadcasted_iota(jnp.int32, (1, 1, w), 2)
    tvec = jnp.where(lane < (w - (_BS - 1)), thresh, 0)
    m = jnp.where(b31 < tvec, 1.0, 0.0)
    mu = jnp.concatenate([m, jnp.zeros((ct, h - hv, w), jnp.float32)], axis=1)
    t = jnp.maximum(mu, _shift_h(mu, 1, ct, h, w))
    t = jnp.maximum(t, _shift_h(t, 2, ct, h, w))
    r = jnp.maximum(t, _shift_h(mu, 4, ct, h, w))
    t = jnp.maximum(r, _shift_w(r, 1, ct, h, w))
    t = jnp.maximum(t, _shift_w(t, 2, ct, h, w))
    d = jnp.maximum(t, _shift_w(r, 4, ct, h, w))
    return d


def _count_body(h, w, inner):
    tiles = _CTC // _ST

    def body(gamma_ref, out_ref, acc_ref):
        p = pl.program_id(0)
        s = pl.program_id(1)
        part = 0.0
        for j in range(tiles):
            d = _dilated_mask(gamma_ref[0, 0], (p * inner + s) * tiles + j, h, w)
            part += jnp.sum(d)

        @pl.when(s == 0)
        def _():
            acc_ref[0, 0] = 0.0

        acc_ref[0, 0] += part

        @pl.when(s == inner - 1)
        def _():
            out_ref[p, 0] = acc_ref[0, 0]

    return body


def _apply_body(h, w, count_m):
    tiles = _CTA // _ST

    def body(gamma_ref, cnt_ref, x_ref, out_ref):
        i = pl.program_id(0)
        sum_dilated = cnt_ref[0, 0] + cnt_ref[1, 0]
        scale = count_m / (count_m - sum_dilated)
        for j in range(tiles):
            d = _dilated_mask(gamma_ref[0, 0], i * tiles + j, h, w)
            sl = pl.ds(j * _ST, _ST)
            out_ref[sl, :, :] = jnp.where(d > 0.5, 0.0, x_ref[sl, :, :] * scale)

    return body


def kernel(x, gamma):
    b, c, h, w = x.shape
    n = b * c
    inner = n // _CTC // _PCORES
    xf = x.reshape(n, h, w)
    g = gamma.reshape(1, 1).astype(jnp.float32)
    count_m = float(b * c * h * w)

    partials = pl.pallas_call(
        _count_body(h, w, inner),
        grid=(_PCORES, inner),
        in_specs=[pl.BlockSpec(memory_space=pltpu.SMEM)],
        out_specs=pl.BlockSpec(memory_space=pltpu.SMEM),
        out_shape=jax.ShapeDtypeStruct((_PCORES, 1), jnp.float32),
        scratch_shapes=[pltpu.SMEM((1, 1), jnp.float32)],
        compiler_params=pltpu.CompilerParams(
            dimension_semantics=("parallel", "arbitrary"),
        ),
    )(g)

    out = pl.pallas_call(
        _apply_body(h, w, count_m),
        grid=(n // _CTA,),
        in_specs=[
            pl.BlockSpec(memory_space=pltpu.SMEM),
            pl.BlockSpec(memory_space=pltpu.SMEM),
            pl.BlockSpec((_CTA, h, w), lambda i: (i, 0, 0)),
        ],
        out_specs=pl.BlockSpec((_CTA, h, w), lambda i: (i, 0, 0)),
        out_shape=jax.ShapeDtypeStruct((n, h, w), jnp.float32),
        compiler_params=pltpu.CompilerParams(
            dimension_semantics=("parallel",),
        ),
    )(g, partials, xf)

    return out.reshape(b, c, h, w)
